# baseline (speedup 1.0000x reference)
import functools

import jax
import jax.numpy as jnp
from jax import lax
from jax.experimental import pallas as pl
from jax.experimental.pallas import tpu as pltpu
from jax.experimental.pallas import tpu_sc as plsc

B = 128
G = 8192
GBITS = 13
NUM_MASKED = 4915
L = 16
CHUNKS = G // L
D0, D1, D2 = 11, 11, 8
NB0, NB1, NB2 = 1 << D0, 1 << D1, 1 << D2
N_WORKERS = 32
ROWS_PER_TILE = B // N_WORKERS
UNROLL = 8

_mesh = plsc.VectorSubcoreMesh(core_axis_name="c", subcore_axis_name="s")

_scratch = []
for _ in range(ROWS_PER_TILE):
    _scratch += [
        pltpu.VMEM((G,), jnp.float32),
        pltpu.VMEM((G,), jnp.int32),
        pltpu.VMEM((G,), jnp.int32),
        pltpu.VMEM((NB0,), jnp.int32),
        pltpu.VMEM((NB1,), jnp.int32),
    ]


@functools.partial(
    pl.kernel,
    out_type=jax.ShapeDtypeStruct((B, G), jnp.int32),
    mesh=_mesh,
    scratch_types=_scratch,
    compiler_params=pltpu.CompilerParams(needs_layout_passes=False),
)
def _argsort_rows(noise_hbm, out_hbm, *scratch):
    noise_v = scratch[0::5]
    buf_a = scratch[1::5]
    buf_b = scratch[2::5]
    hist_a = scratch[3::5]
    hist_b = scratch[4::5]

    core = lax.axis_index("c")
    sub = lax.axis_index("s")
    wid = sub * 2 + core
    base_row = wid * ROWS_PER_TILE
    iota = lax.iota(jnp.int32, L)
    NWAY = ROWS_PER_TILE

    cnt0, _ = plsc.scan_count(jnp.zeros((L,), jnp.int32))
    c0 = jnp.min(cnt0)

    for q in range(NWAY):
        pltpu.sync_copy(noise_hbm.at[base_row + q], noise_v[q])

    ones = jnp.ones((L,), jnp.int32)
    zeros = jnp.zeros((L,), jnp.int32)

    def clear(refs, n):
        def clr(i, carry):
            for ref in refs:
                ref[pl.ds(i * L, L)] = zeros
            return carry
        lax.fori_loop(0, n // L, clr, jnp.int32(0), unroll=UNROLL)

    def excl_scan(refs, n, name):
        def scan(i, carry):
            nxt = []
            for k, ref in enumerate(refs):
                v = ref[pl.ds(i * L, L)]
                incl = plsc.cumsum(v)
                ref[pl.ds(i * L, L)] = incl - v + carry[k]
                nxt.append(carry[k] + jnp.max(incl))
            return tuple(nxt)
        with jax.named_scope(name):
            lax.fori_loop(0, n // L, scan, (jnp.int32(0),) * len(refs))

    with jax.named_scope("clr0"):
        clear(hist_a, NB0)

    def histo(c, carry):
        kvs = [plsc.bitcast(noise_v[q][pl.ds(c * L, L)], jnp.int32)
               for q in range(NWAY)]
        for q in range(NWAY):
            plsc.addupdate_scatter(hist_a[q], [kvs[q] & (NB0 - 1)], ones)
        return carry

    with jax.named_scope("histo0"):
        lax.fori_loop(0, CHUNKS, histo, jnp.int32(0), unroll=UNROLL)

    excl_scan(hist_a, NB0, "scan0")
    with jax.named_scope("clrB"):
        clear(hist_b, NB1)

    def scat0(c, carry):
        loaded = []
        for q in range(NWAY):
            kv = plsc.bitcast(noise_v[q][pl.ds(c * L, L)], jnp.int32)
            d = kv & (NB0 - 1)
            pay = lax.shift_left(lax.shift_right_logical(kv, D0), GBITS) \
                | (c * L + iota)
            loaded.append((d, pay))
        cnts = [plsc.scan_count(d)[0] for d, _ in loaded]
        for q in range(NWAY):
            d, pay = loaded[q]
            starts = plsc.load_gather(hist_a[q], [d])
            plsc.store_scatter(buf_a[q], [starts + cnts[q] - c0], pay)
            plsc.addupdate_scatter(hist_a[q], [d], ones)
            d1 = lax.shift_right_logical(pay, GBITS) & (NB1 - 1)
            plsc.addupdate_scatter(hist_b[q], [d1], ones)
        return carry

    with jax.named_scope("scat0"):
        lax.fori_loop(0, CHUNKS, scat0, jnp.int32(0), unroll=UNROLL)

    excl_scan(hist_b, NB1, "scan1")
    with jax.named_scope("clrA2"):
        clear(hist_a, NB2)

    def scat1(c, carry):
        pays = [buf_a[q][pl.ds(c * L, L)] for q in range(NWAY)]
        ds = [lax.shift_right_logical(pay, GBITS) & (NB1 - 1) for pay in pays]
        cnts = [plsc.scan_count(d)[0] for d in ds]
        for q in range(NWAY):
            starts = plsc.load_gather(hist_b[q], [ds[q]])
            plsc.store_scatter(buf_b[q], [starts + cnts[q] - c0], pays[q])
            plsc.addupdate_scatter(hist_b[q], [ds[q]], ones)
            d2 = lax.shift_right_logical(pays[q], GBITS + D1)
            plsc.addupdate_scatter(hist_a[q], [d2], ones)
        return carry

    with jax.named_scope("scat1"):
        lax.fori_loop(0, CHUNKS, scat1, jnp.int32(0), unroll=UNROLL)

    excl_scan(hist_a, NB2, "scan2")

    def scat2(c, carry):
        pays = [buf_b[q][pl.ds(c * L, L)] for q in range(NWAY)]
        ds = [lax.shift_right_logical(pay, GBITS + D1) for pay in pays]
        cnts = [plsc.scan_count(d)[0] for d in ds]
        for q in range(NWAY):
            starts = plsc.load_gather(hist_a[q], [ds[q]])
            plsc.store_scatter(buf_a[q], [starts + cnts[q] - c0],
                               pays[q] & (G - 1))
            plsc.addupdate_scatter(hist_a[q], [ds[q]], ones)
        return carry

    with jax.named_scope("scat2"):
        lax.fori_loop(0, CHUNKS, scat2, jnp.int32(0), unroll=UNROLL)

    for q in range(NWAY):
        pltpu.sync_copy(buf_a[q], out_hbm.at[base_row + q])


def kernel(x, noise):
    del x
    perm = _argsort_rows(noise)
    return perm[:, :NUM_MASKED], perm[:, NUM_MASKED:]


# speedup vs baseline: 1.0183x; 1.0183x over previous
import functools

import jax
import jax.numpy as jnp
from jax import lax
from jax.experimental import pallas as pl
from jax.experimental.pallas import tpu as pltpu
from jax.experimental.pallas import tpu_sc as plsc

B = 128
G = 8192
GBITS = 13
NUM_MASKED = 4915
L = 16
CHUNKS = G // L
D0, D1, D2 = 11, 11, 8
NB0, NB1, NB2 = 1 << D0, 1 << D1, 1 << D2
N_WORKERS = 32
ROWS_PER_TILE = B // N_WORKERS
UNROLL = 4

_mesh = plsc.VectorSubcoreMesh(core_axis_name="c", subcore_axis_name="s")

_scratch = []
for _ in range(ROWS_PER_TILE):
    _scratch += [
        pltpu.VMEM((G,), jnp.float32),
        pltpu.VMEM((OUT_WORDS,), jnp.int32),
        pltpu.VMEM((G,), jnp.int32),
        pltpu.VMEM((NB0,), jnp.int32),
        pltpu.VMEM((NB1,), jnp.int32),
    ]


@functools.partial(
    pl.kernel,
    out_type=jax.ShapeDtypeStruct((B, G), jnp.int32),
    mesh=_mesh,
    scratch_types=_scratch,
    compiler_params=pltpu.CompilerParams(needs_layout_passes=False),
)
def _argsort_rows(noise_hbm, out_hbm, *scratch):
    noise_v = scratch[0::5]
    buf_a = scratch[1::5]
    buf_b = scratch[2::5]
    hist_a = scratch[3::5]
    hist_b = scratch[4::5]

    core = lax.axis_index("c")
    sub = lax.axis_index("s")
    wid = sub * 2 + core
    base_row = wid * ROWS_PER_TILE
    iota = lax.iota(jnp.int32, L)
    NWAY = ROWS_PER_TILE

    cnt0, _ = plsc.scan_count(jnp.zeros((L,), jnp.int32))
    c0 = jnp.min(cnt0)

    for q in range(NWAY):
        pltpu.sync_copy(noise_hbm.at[base_row + q], noise_v[q])

    ones = jnp.ones((L,), jnp.int32)
    zeros = jnp.zeros((L,), jnp.int32)

    def clear(refs, n):
        def clr(i, carry):
            for ref in refs:
                ref[pl.ds(i * L, L)] = zeros
            return carry
        lax.fori_loop(0, n // L, clr, jnp.int32(0), unroll=UNROLL)

    def excl_scan(refs, n, name):
        def scan(i, carry):
            nxt = []
            for k, ref in enumerate(refs):
                v = ref[pl.ds(i * L, L)]
                incl = plsc.cumsum(v)
                ref[pl.ds(i * L, L)] = incl - v + carry[k]
                nxt.append(carry[k] + jnp.max(incl))
            return tuple(nxt)
        with jax.named_scope(name):
            lax.fori_loop(0, n // L, scan, (jnp.int32(0) - c0,) * len(refs))

    with jax.named_scope("clr0"):
        clear(hist_a, NB0)

    def histo(c, carry):
        kvs = [plsc.bitcast(noise_v[q][pl.ds(c * L, L)], jnp.int32)
               for q in range(NWAY)]
        for q in range(NWAY):
            plsc.addupdate_scatter(hist_a[q], [kvs[q] & (NB0 - 1)], ones)
        return carry

    with jax.named_scope("histo0"):
        lax.fori_loop(0, CHUNKS, histo, jnp.int32(0), unroll=UNROLL)

    excl_scan(hist_a, NB0, "scan0")
    with jax.named_scope("clrB"):
        clear(hist_b, NB1)

    def scat0(c, carry):
        loaded = []
        for q in range(NWAY):
            kv = plsc.bitcast(noise_v[q][pl.ds(c * L, L)], jnp.int32)
            d = kv & (NB0 - 1)
            pay = lax.shift_left(lax.shift_right_logical(kv, D0), GBITS) \
                | (c * L + iota)
            loaded.append((d, pay))
        cnts = [plsc.scan_count(d)[0] for d, _ in loaded]
        for q in range(NWAY):
            d, pay = loaded[q]
            starts = plsc.load_gather(hist_a[q], [d])
            plsc.store_scatter(buf_a[q], [starts + cnts[q]], pay)
            plsc.addupdate_scatter(hist_a[q], [d], ones)
            d1 = lax.shift_right_logical(pay, GBITS) & (NB1 - 1)
            plsc.addupdate_scatter(hist_b[q], [d1], ones)
        return carry

    with jax.named_scope("scat0"):
        lax.fori_loop(0, CHUNKS, scat0, jnp.int32(0), unroll=UNROLL)

    excl_scan(hist_b, NB1, "scan1")
    with jax.named_scope("clrA2"):
        clear(hist_a, NB2)

    def scat1(c, carry):
        pays = [buf_a[q][pl.ds(c * L, L)] for q in range(NWAY)]
        ds = [lax.shift_right_logical(pay, GBITS) & (NB1 - 1) for pay in pays]
        cnts = [plsc.scan_count(d)[0] for d in ds]
        for q in range(NWAY):
            starts = plsc.load_gather(hist_b[q], [ds[q]])
            plsc.store_scatter(buf_b[q], [starts + cnts[q]], pays[q])
            plsc.addupdate_scatter(hist_b[q], [ds[q]], ones)
            d2 = lax.shift_right_logical(pays[q], GBITS + D1)
            plsc.addupdate_scatter(hist_a[q], [d2], ones)
        return carry

    with jax.named_scope("scat1"):
        lax.fori_loop(0, CHUNKS, scat1, jnp.int32(0), unroll=UNROLL)

    excl_scan(hist_a, NB2, "scan2")

    def scat2(c, carry):
        pays = [buf_b[q][pl.ds(c * L, L)] for q in range(NWAY)]
        ds = [lax.shift_right_logical(pay, GBITS + D1) for pay in pays]
        cnts = [plsc.scan_count(d)[0] for d in ds]
        for q in range(NWAY):
            starts = plsc.load_gather(hist_a[q], [ds[q]])
            plsc.store_scatter(buf_a[q], [starts + cnts[q]],
                               pays[q] & (G - 1))
            plsc.addupdate_scatter(hist_a[q], [ds[q]], ones)
        return carry

    with jax.named_scope("scat2"):
        lax.fori_loop(0, CHUNKS, scat2, jnp.int32(0), unroll=UNROLL)

    for q in range(NWAY):
        pltpu.sync_copy(buf_a[q].at[pl.ds(0, G)], out_hbm.at[base_row + q])


def kernel(x, noise):
    del x
    perm = _argsort_rows(noise)
    return perm[:, :NUM_MASKED], perm[:, NUM_MASKED:]


# speedup vs baseline: 1.0372x; 1.0185x over previous
import functools

import jax
import jax.numpy as jnp
from jax import lax
from jax.experimental import pallas as pl
from jax.experimental.pallas import tpu as pltpu
from jax.experimental.pallas import tpu_sc as plsc

B = 128
G = 8192
GBITS = 13
NUM_MASKED = 4915
L = 16
CHUNKS = G // L
D0, D1, D2 = 11, 11, 8
NB0, NB1, NB2 = 1 << D0, 1 << D1, 1 << D2
N_WORKERS = 32
ROWS_PER_TILE = B // N_WORKERS
UNROLL = 4

_mesh = plsc.VectorSubcoreMesh(core_axis_name="c", subcore_axis_name="s")

_scratch = []
for _ in range(ROWS_PER_TILE):
    _scratch += [
        pltpu.VMEM((G,), jnp.float32),
        pltpu.VMEM((OUT_WORDS,), jnp.int32),
        pltpu.VMEM((G,), jnp.int32),
        pltpu.VMEM((NB0,), jnp.int32),
        pltpu.VMEM((NB1,), jnp.int32),
    ]
_scratch.append(pltpu.SemaphoreType.DMA)


@functools.partial(
    pl.kernel,
    out_type=jax.ShapeDtypeStruct((B, G), jnp.int32),
    mesh=_mesh,
    scratch_types=_scratch,
    compiler_params=pltpu.CompilerParams(needs_layout_passes=False),
)
def _argsort_rows(noise_hbm, out_hbm, *scratch):
    sem = scratch[-1]
    scratch = scratch[:-1]
    noise_v = scratch[0::5]
    buf_a = scratch[1::5]
    buf_b = scratch[2::5]
    hist_a = scratch[3::5]
    hist_b = scratch[4::5]

    core = lax.axis_index("c")
    sub = lax.axis_index("s")
    wid = sub * 2 + core
    base_row = wid * ROWS_PER_TILE
    iota = lax.iota(jnp.int32, L)
    NWAY = ROWS_PER_TILE

    cnt0, _ = plsc.scan_count(jnp.zeros((L,), jnp.int32))
    c0 = jnp.min(cnt0)

    copies = [pltpu.async_copy(noise_hbm.at[base_row + q], noise_v[q], sem)
              for q in range(NWAY)]
    for cp in copies:
        cp.wait()

    ones = jnp.ones((L,), jnp.int32)
    zeros = jnp.zeros((L,), jnp.int32)

    def clear(refs, n):
        def clr(i, carry):
            for ref in refs:
                ref[pl.ds(i * L, L)] = zeros
            return carry
        lax.fori_loop(0, n // L, clr, jnp.int32(0), unroll=UNROLL)

    def excl_scan(refs, n, name):
        def scan(i, carry):
            nxt = []
            for k, ref in enumerate(refs):
                v = ref[pl.ds(i * L, L)]
                incl = plsc.cumsum(v)
                ref[pl.ds(i * L, L)] = incl - v + carry[k]
                nxt.append(carry[k] + jnp.max(incl))
            return tuple(nxt)
        with jax.named_scope(name):
            lax.fori_loop(0, n // L, scan, (jnp.int32(0) - c0,) * len(refs))

    with jax.named_scope("clr0"):
        clear(hist_a, NB0)

    def histo(c, carry):
        kvs = [plsc.bitcast(noise_v[q][pl.ds(c * L, L)], jnp.int32)
               for q in range(NWAY)]
        for q in range(NWAY):
            plsc.addupdate_scatter(hist_a[q], [kvs[q] & (NB0 - 1)], ones)
        return carry

    with jax.named_scope("histo0"):
        lax.fori_loop(0, CHUNKS, histo, jnp.int32(0), unroll=UNROLL)

    excl_scan(hist_a, NB0, "scan0")
    with jax.named_scope("clrB"):
        clear(hist_b, NB1)

    def scat0(c, carry):
        loaded = []
        for q in range(NWAY):
            kv = plsc.bitcast(noise_v[q][pl.ds(c * L, L)], jnp.int32)
            d = kv & (NB0 - 1)
            pay = lax.shift_left(lax.shift_right_logical(kv, D0), GBITS) \
                | (c * L + iota)
            loaded.append((d, pay))
        cnts = [plsc.scan_count(d)[0] for d, _ in loaded]
        for q in range(NWAY):
            d, pay = loaded[q]
            starts = plsc.load_gather(hist_a[q], [d])
            plsc.store_scatter(buf_a[q], [starts + cnts[q]], pay)
            plsc.addupdate_scatter(hist_a[q], [d], ones)
            d1 = lax.shift_right_logical(pay, GBITS) & (NB1 - 1)
            plsc.addupdate_scatter(hist_b[q], [d1], ones)
        return carry

    with jax.named_scope("scat0"):
        lax.fori_loop(0, CHUNKS, scat0, jnp.int32(0), unroll=UNROLL)

    excl_scan(hist_b, NB1, "scan1")
    with jax.named_scope("clrA2"):
        clear(hist_a, NB2)

    def scat1(c, carry):
        pays = [buf_a[q][pl.ds(c * L, L)] for q in range(NWAY)]
        ds = [lax.shift_right_logical(pay, GBITS) & (NB1 - 1) for pay in pays]
        cnts = [plsc.scan_count(d)[0] for d in ds]
        for q in range(NWAY):
            starts = plsc.load_gather(hist_b[q], [ds[q]])
            plsc.store_scatter(buf_b[q], [starts + cnts[q]], pays[q])
            plsc.addupdate_scatter(hist_b[q], [ds[q]], ones)
            d2 = lax.shift_right_logical(pays[q], GBITS + D1)
            plsc.addupdate_scatter(hist_a[q], [d2], ones)
        return carry

    with jax.named_scope("scat1"):
        lax.fori_loop(0, CHUNKS, scat1, jnp.int32(0), unroll=UNROLL)

    excl_scan(hist_a, NB2, "scan2")

    def scat2(c, carry):
        pays = [buf_b[q][pl.ds(c * L, L)] for q in range(NWAY)]
        ds = [lax.shift_right_logical(pay, GBITS + D1) for pay in pays]
        cnts = [plsc.scan_count(d)[0] for d in ds]
        for q in range(NWAY):
            starts = plsc.load_gather(hist_a[q], [ds[q]])
            plsc.store_scatter(buf_a[q], [starts + cnts[q]],
                               pays[q] & (G - 1))
            plsc.addupdate_scatter(hist_a[q], [ds[q]], ones)
        return carry

    with jax.named_scope("scat2"):
        lax.fori_loop(0, CHUNKS, scat2, jnp.int32(0), unroll=UNROLL)

    copies = [pltpu.async_copy(buf_a[q].at[pl.ds(0, G)],
                               out_hbm.at[base_row + q], sem)
              for q in range(NWAY)]
    for cp in copies:
        cp.wait()


def kernel(x, noise):
    del x
    perm = _argsort_rows(noise)
    return perm[:, :NUM_MASKED], perm[:, NUM_MASKED:]


# speedup vs baseline: 1.0394x; 1.0021x over previous
import functools

import jax
import jax.numpy as jnp
from jax import lax
from jax.experimental import pallas as pl
from jax.experimental.pallas import tpu as pltpu
from jax.experimental.pallas import tpu_sc as plsc

B = 128
G = 8192
GBITS = 13
NUM_MASKED = 4915
L = 16
CHUNKS = G // L
D0, D1, D2 = 11, 11, 8
NB0, NB1, NB2 = 1 << D0, 1 << D1, 1 << D2
N_WORKERS = 32
ROWS_PER_TILE = B // N_WORKERS
UNROLL = 4

_mesh = plsc.VectorSubcoreMesh(core_axis_name="c", subcore_axis_name="s")

_scratch = []
for _ in range(ROWS_PER_TILE):
    _scratch += [
        pltpu.VMEM((G,), jnp.float32),
        pltpu.VMEM((G,), jnp.int32),
        pltpu.VMEM((G,), jnp.int32),
        pltpu.VMEM((NB0,), jnp.int32),
        pltpu.VMEM((NB1,), jnp.int32),
    ]
_scratch.append(pltpu.SemaphoreType.DMA)


@functools.partial(
    pl.kernel,
    out_type=jax.ShapeDtypeStruct((B, G), jnp.int32),
    mesh=_mesh,
    scratch_types=_scratch,
    compiler_params=pltpu.CompilerParams(needs_layout_passes=False),
)
def _argsort_rows(noise_hbm, out_hbm, *scratch):
    sem = scratch[-1]
    scratch = scratch[:-1]
    noise_v = scratch[0::5]
    buf_a = scratch[1::5]
    buf_b = scratch[2::5]
    hist_a = scratch[3::5]
    hist_b = scratch[4::5]

    core = lax.axis_index("c")
    sub = lax.axis_index("s")
    wid = sub * 2 + core
    base_row = wid * ROWS_PER_TILE
    iota = lax.iota(jnp.int32, L)
    NWAY = ROWS_PER_TILE

    cnt0, _ = plsc.scan_count(jnp.zeros((L,), jnp.int32))
    c0 = jnp.min(cnt0)

    copies = [pltpu.async_copy(noise_hbm.at[base_row + q], noise_v[q], sem)
              for q in range(NWAY)]
    for cp in copies:
        cp.wait()

    ones = jnp.ones((L,), jnp.int32)
    zeros = jnp.zeros((L,), jnp.int32)

    def clear(refs, n):
        def clr(i, carry):
            for ref in refs:
                ref[pl.ds(i * L, L)] = zeros
            return carry
        lax.fori_loop(0, n // L, clr, jnp.int32(0), unroll=UNROLL)

    def excl_scan(refs, n):
        def scan(i, carry):
            nxt = []
            for k, ref in enumerate(refs):
                v = ref[pl.ds(i * L, L)]
                incl = plsc.cumsum(v)
                ref[pl.ds(i * L, L)] = incl - v + carry[k]
                nxt.append(carry[k] + jnp.max(incl))
            return tuple(nxt)
        lax.fori_loop(0, n // L, scan, (jnp.int32(0) - c0,) * len(refs))

    clear(hist_a, NB0)

    def histo(c, carry):
        kvs = [plsc.bitcast(noise_v[q][pl.ds(c * L, L)], jnp.int32)
               for q in range(NWAY)]
        for q in range(NWAY):
            plsc.addupdate_scatter(hist_a[q], [kvs[q] & (NB0 - 1)], ones)
        return carry

    lax.fori_loop(0, CHUNKS, histo, jnp.int32(0), unroll=UNROLL)

    excl_scan(hist_a, NB0)
    clear(hist_b, NB1)

    def scat0(c, carry):
        loaded = []
        for q in range(NWAY):
            kv = plsc.bitcast(noise_v[q][pl.ds(c * L, L)], jnp.int32)
            d = kv & (NB0 - 1)
            pay = lax.shift_left(lax.shift_right_logical(kv, D0), GBITS) \
                | (c * L + iota)
            loaded.append((d, pay))
        cnts = [plsc.scan_count(d)[0] for d, _ in loaded]
        for q in range(NWAY):
            d, pay = loaded[q]
            starts = plsc.load_gather(hist_a[q], [d])
            plsc.store_scatter(buf_a[q], [starts + cnts[q]], pay)
            plsc.addupdate_scatter(hist_a[q], [d], ones)
            d1 = lax.shift_right_logical(pay, GBITS) & (NB1 - 1)
            plsc.addupdate_scatter(hist_b[q], [d1], ones)
        return carry

    lax.fori_loop(0, CHUNKS, scat0, jnp.int32(0), unroll=UNROLL)

    excl_scan(hist_b, NB1)
    clear(hist_a, NB2)

    def scat1(c, carry):
        pays = [buf_a[q][pl.ds(c * L, L)] for q in range(NWAY)]
        ds = [lax.shift_right_logical(pay, GBITS) & (NB1 - 1) for pay in pays]
        cnts = [plsc.scan_count(d)[0] for d in ds]
        for q in range(NWAY):
            starts = plsc.load_gather(hist_b[q], [ds[q]])
            plsc.store_scatter(buf_b[q], [starts + cnts[q]], pays[q])
            plsc.addupdate_scatter(hist_b[q], [ds[q]], ones)
            d2 = lax.shift_right_logical(pays[q], GBITS + D1)
            plsc.addupdate_scatter(hist_a[q], [d2], ones)
        return carry

    lax.fori_loop(0, CHUNKS, scat1, jnp.int32(0), unroll=UNROLL)

    excl_scan(hist_a, NB2)

    def scat2(c, carry):
        pays = [buf_b[q][pl.ds(c * L, L)] for q in range(NWAY)]
        ds = [lax.shift_right_logical(pay, GBITS + D1) for pay in pays]
        cnts = [plsc.scan_count(d)[0] for d in ds]
        for q in range(NWAY):
            starts = plsc.load_gather(hist_a[q], [ds[q]])
            plsc.store_scatter(buf_a[q], [starts + cnts[q]],
                               pays[q] & (G - 1))
            plsc.addupdate_scatter(hist_a[q], [ds[q]], ones)
        return carry

    lax.fori_loop(0, CHUNKS, scat2, jnp.int32(0), unroll=UNROLL)

    copies = [pltpu.async_copy(buf_a[q],
                               out_hbm.at[base_row + q], sem)
              for q in range(NWAY)]
    for cp in copies:
        cp.wait()


def kernel(x, noise):
    del x
    perm = _argsort_rows(noise)
    return perm[:, :NUM_MASKED], perm[:, NUM_MASKED:]


# speedup vs baseline: 1.0546x; 1.0147x over previous
import functools

import jax
import jax.numpy as jnp
from jax import lax
from jax.experimental import pallas as pl
from jax.experimental.pallas import tpu as pltpu
from jax.experimental.pallas import tpu_sc as plsc

B = 128
G = 8192
GBITS = 13
NUM_MASKED = 4915
L = 16
CHUNKS = G // L
D0, D1, D2 = 11, 11, 8
NB0, NB1, NB2 = 1 << D0, 1 << D1, 1 << D2
N_WORKERS = 32
ROWS_PER_TILE = B // N_WORKERS
UNROLL = 4

_mesh = plsc.VectorSubcoreMesh(core_axis_name="c", subcore_axis_name="s")

_scratch = []
for _ in range(ROWS_PER_TILE):
    _scratch += [
        pltpu.VMEM((G,), jnp.float32),
        pltpu.VMEM((G,), jnp.int32),
        pltpu.VMEM((G,), jnp.int32),
        pltpu.VMEM((NB0,), jnp.int32),
        pltpu.VMEM((NB1,), jnp.int32),
    ]
_scratch.append(pltpu.SemaphoreType.DMA)


@functools.partial(
    pl.kernel,
    out_type=jax.ShapeDtypeStruct((B, G), jnp.int32),
    mesh=_mesh,
    scratch_types=_scratch,
    compiler_params=pltpu.CompilerParams(needs_layout_passes=False),
)
def _argsort_rows(noise_hbm, out_hbm, *scratch):
    sem = scratch[-1]
    scratch = scratch[:-1]
    noise_v = scratch[0::5]
    buf_a = scratch[1::5]
    buf_b = scratch[2::5]
    hist_a = scratch[3::5]
    hist_b = scratch[4::5]

    core = lax.axis_index("c")
    sub = lax.axis_index("s")
    wid = sub * 2 + core
    base_row = wid * ROWS_PER_TILE
    iota = lax.iota(jnp.int32, L)
    NWAY = ROWS_PER_TILE

    cnt0, _ = plsc.scan_count(jnp.zeros((L,), jnp.int32))
    c0 = jnp.min(cnt0)

    copies = [pltpu.async_copy(noise_hbm.at[base_row + q], noise_v[q], sem)
              for q in range(NWAY)]
    for cp in copies:
        cp.wait()

    ones = jnp.ones((L,), jnp.int32)
    zeros = jnp.zeros((L,), jnp.int32)

    def clear(refs, n):
        def clr(i, carry):
            for ref in refs:
                ref[pl.ds(i * L, L)] = zeros
            return carry
        lax.fori_loop(0, n // L, clr, jnp.int32(0), unroll=UNROLL)

    def excl_scan(refs, n):
        def scan(i, carry):
            nxt = []
            for k, ref in enumerate(refs):
                v = ref[pl.ds(i * L, L)]
                incl = plsc.cumsum(v)
                ref[pl.ds(i * L, L)] = incl - v + carry[k]
                nxt.append(carry[k] + jnp.max(incl))
            return tuple(nxt)
        lax.fori_loop(0, n // L, scan, (jnp.int32(0) - c0,) * len(refs))

    clear(hist_a, NB0)

    @plsc.parallel_loop(0, CHUNKS, unroll=UNROLL)
    def histo(c):
        kvs = [plsc.bitcast(noise_v[q][pl.ds(c * L, L)], jnp.int32)
               for q in range(NWAY)]
        for q in range(NWAY):
            plsc.addupdate_scatter(hist_a[q], [kvs[q] & (NB0 - 1)], ones)

    excl_scan(hist_a, NB0)
    clear(hist_b, NB1)

    def scat0(c, carry):
        loaded = []
        for q in range(NWAY):
            kv = plsc.bitcast(noise_v[q][pl.ds(c * L, L)], jnp.int32)
            d = kv & (NB0 - 1)
            pay = lax.shift_left(lax.shift_right_logical(kv, D0), GBITS) \
                | (c * L + iota)
            loaded.append((d, pay))
        cnts = [plsc.scan_count(d)[0] for d, _ in loaded]
        for q in range(NWAY):
            d, pay = loaded[q]
            starts = plsc.load_gather(hist_a[q], [d])
            plsc.store_scatter(buf_a[q], [starts + cnts[q]], pay)
            plsc.addupdate_scatter(hist_a[q], [d], ones)
            d1 = lax.shift_right_logical(pay, GBITS) & (NB1 - 1)
            plsc.addupdate_scatter(hist_b[q], [d1], ones)
        return carry

    lax.fori_loop(0, CHUNKS, scat0, jnp.int32(0), unroll=UNROLL)

    excl_scan(hist_b, NB1)
    clear(hist_a, NB2)

    def scat1(c, carry):
        pays = [buf_a[q][pl.ds(c * L, L)] for q in range(NWAY)]
        ds = [lax.shift_right_logical(pay, GBITS) & (NB1 - 1) for pay in pays]
        cnts = [plsc.scan_count(d)[0] for d in ds]
        for q in range(NWAY):
            starts = plsc.load_gather(hist_b[q], [ds[q]])
            plsc.store_scatter(buf_b[q], [starts + cnts[q]], pays[q])
            plsc.addupdate_scatter(hist_b[q], [ds[q]], ones)
            d2 = lax.shift_right_logical(pays[q], GBITS + D1)
            plsc.addupdate_scatter(hist_a[q], [d2], ones)
        return carry

    lax.fori_loop(0, CHUNKS, scat1, jnp.int32(0), unroll=UNROLL)

    excl_scan(hist_a, NB2)

    def scat2(c, carry):
        pays = [buf_b[q][pl.ds(c * L, L)] for q in range(NWAY)]
        ds = [lax.shift_right_logical(pay, GBITS + D1) for pay in pays]
        cnts = [plsc.scan_count(d)[0] for d in ds]
        for q in range(NWAY):
            starts = plsc.load_gather(hist_a[q], [ds[q]])
            plsc.store_scatter(buf_a[q], [starts + cnts[q]],
                               pays[q] & (G - 1))
            plsc.addupdate_scatter(hist_a[q], [ds[q]], ones)
        return carry

    lax.fori_loop(0, CHUNKS, scat2, jnp.int32(0), unroll=UNROLL)

    copies = [pltpu.async_copy(buf_a[q],
                               out_hbm.at[base_row + q], sem)
              for q in range(NWAY)]
    for cp in copies:
        cp.wait()


def kernel(x, noise):
    del x
    perm = _argsort_rows(noise)
    return perm[:, :NUM_MASKED], perm[:, NUM_MASKED:]
